# baseline (speedup 1.0000x reference)
import functools

import jax
import jax.numpy as jnp
from jax import lax
from jax.experimental import pallas as pl
from jax.experimental.pallas import tpu as pltpu
from jax.experimental.pallas import tpu_sc as plsc

VOCAB = 100000
EMB = 32
N = 20
BATCH = 1024
TOTAL = BATCH * N
K = N * EMB

NC = 2
NS = 16
NW = NC * NS
ROWS_PER_W = TOTAL // NW

V_TILE = 512
GRID_V = (VOCAB + V_TILE - 1) // V_TILE


def _gather_body(table_hbm, idx_hbm, out_hbm, idx_v, rows_v, sem):
    wid = lax.axis_index("s") * NC + lax.axis_index("c")
    base = wid * ROWS_PER_W
    pltpu.sync_copy(idx_hbm.at[pl.ds(base, ROWS_PER_W)], idx_v)
    pltpu.async_copy(table_hbm.at[idx_v], rows_v, sem).wait()
    pltpu.sync_copy(rows_v, out_hbm.at[pl.ds(base, ROWS_PER_W)])


_sc_gather = functools.partial(
    pl.kernel,
    mesh=plsc.VectorSubcoreMesh(core_axis_name="c", subcore_axis_name="s"),
    out_type=jax.ShapeDtypeStruct((TOTAL, EMB), jnp.float32),
    compiler_params=pltpu.CompilerParams(use_tc_tiling_on_sc=False),
    scratch_types=[
        pltpu.VMEM((ROWS_PER_W,), jnp.int32),
        pltpu.VMEM((ROWS_PER_W, EMB), jnp.float32),
        pltpu.SemaphoreType.DMA,
    ],
)(_gather_body)


def _matmul_body(flat_ref, w_ref, b_ref, out_ref):
    acc = lax.dot_general(
        flat_ref[...].astype(jnp.bfloat16),
        w_ref[...].astype(jnp.bfloat16),
        dimension_numbers=(((1,), (1,)), ((), ())),
        preferred_element_type=jnp.float32,
    )
    out_ref[...] = acc + b_ref[...]


def _projection(flat, W, b2d):
    return pl.pallas_call(
        _matmul_body,
        grid=(GRID_V,),
        in_specs=[
            pl.BlockSpec((BATCH, K), lambda j: (0, 0)),
            pl.BlockSpec((V_TILE, K), lambda j: (j, 0)),
            pl.BlockSpec((1, V_TILE), lambda j: (0, j)),
        ],
        out_specs=pl.BlockSpec((BATCH, V_TILE), lambda j: (0, j)),
        out_shape=jax.ShapeDtypeStruct((BATCH, VOCAB), jnp.float32),
        compiler_params=pltpu.CompilerParams(
            dimension_semantics=("arbitrary",),
        ),
    )(flat, W, b2d)


def kernel(inputs, emb_table, W, b):
    idx = inputs[0].reshape(TOTAL)
    flat = _sc_gather(emb_table, idx).reshape(BATCH, K)
    return _projection(flat, W, b.reshape(1, VOCAB))


# speedup vs baseline: 1.1571x; 1.1571x over previous
import functools

import jax
import jax.numpy as jnp
from jax import lax
from jax.experimental import pallas as pl
from jax.experimental.pallas import tpu as pltpu
from jax.experimental.pallas import tpu_sc as plsc

VOCAB = 100000
EMB = 32
N = 20
BATCH = 1024
TOTAL = BATCH * N
K = N * EMB

NC = 2
NS = 16
NW = NC * NS
ROWS_PER_W = TOTAL // NW

V_TILE = 2048
GRID_V = (VOCAB + V_TILE - 1) // V_TILE


def _gather_body(table_hbm, idx_hbm, out_hbm, idx_v, rows_v, sem):
    wid = lax.axis_index("s") * NC + lax.axis_index("c")
    base = wid * ROWS_PER_W
    pltpu.sync_copy(idx_hbm.at[pl.ds(base, ROWS_PER_W)], idx_v)
    pltpu.async_copy(table_hbm.at[idx_v], rows_v, sem).wait()
    pltpu.sync_copy(rows_v, out_hbm.at[pl.ds(base, ROWS_PER_W)])


_sc_gather = functools.partial(
    pl.kernel,
    mesh=plsc.VectorSubcoreMesh(core_axis_name="c", subcore_axis_name="s"),
    out_type=jax.ShapeDtypeStruct((TOTAL, EMB), jnp.float32),
    compiler_params=pltpu.CompilerParams(use_tc_tiling_on_sc=False),
    scratch_types=[
        pltpu.VMEM((ROWS_PER_W,), jnp.int32),
        pltpu.VMEM((ROWS_PER_W, EMB), jnp.float32),
        pltpu.SemaphoreType.DMA,
    ],
)(_gather_body)


def _matmul_body(flat_ref, w_ref, b_ref, out_ref):
    acc = lax.dot_general(
        flat_ref[...].astype(jnp.bfloat16),
        w_ref[...].astype(jnp.bfloat16),
        dimension_numbers=(((1,), (1,)), ((), ())),
        preferred_element_type=jnp.float32,
    )
    out_ref[...] = acc + b_ref[...]


def _projection(flat, W, b2d):
    return pl.pallas_call(
        _matmul_body,
        grid=(GRID_V,),
        in_specs=[
            pl.BlockSpec(memory_space=pltpu.MemorySpace.VMEM),
            pl.BlockSpec((V_TILE, K), lambda j: (j, 0)),
            pl.BlockSpec((1, V_TILE), lambda j: (0, j)),
        ],
        out_specs=pl.BlockSpec((BATCH, V_TILE), lambda j: (0, j)),
        out_shape=jax.ShapeDtypeStruct((BATCH, VOCAB), jnp.float32),
        compiler_params=pltpu.CompilerParams(
            dimension_semantics=("arbitrary",),
        ),
    )(flat, W, b2d)


def kernel(inputs, emb_table, W, b):
    idx = inputs[0].reshape(TOTAL)
    flat = _sc_gather(emb_table, idx).reshape(BATCH, K)
    return _projection(flat, W, b.reshape(1, VOCAB))
